# Initial kernel scaffold; baseline (speedup 1.0000x reference)
import functools

import jax
import jax.numpy as jnp
from jax import lax
from jax.experimental import pallas as pl
from jax.experimental.pallas import tpu as pltpu
from jax.experimental.pallas import tpu_sc as plsc

NUM_CORES = 2
NUM_SUBCORES = 16
NW = NUM_CORES * NUM_SUBCORES

CHUNK = 3200


def _build(N, V, D):
    n_per_w = N // NW
    n_chunks = n_per_w // CHUNK
    assert n_per_w % CHUNK == 0

    mesh = plsc.VectorSubcoreMesh(core_axis_name="c", subcore_axis_name="s")

    @functools.partial(
        pl.kernel,
        out_type=jax.ShapeDtypeStruct((N, D), jnp.float32),
        mesh=mesh,
        scratch_types=[
            pltpu.VMEM((CHUNK,), jnp.int32),
            pltpu.VMEM((CHUNK, D), jnp.float32),
            pltpu.SemaphoreType.DMA,
        ],
    )
    def gather_kernel(idx_hbm, table_hbm, out_hbm, idx_v, rows_v, sem):
        wid = lax.axis_index("s") * NUM_CORES + lax.axis_index("c")
        base = wid * n_per_w

        @pl.loop(0, n_chunks)
        def _(i):
            off = base + i * CHUNK
            pltpu.sync_copy(idx_hbm.at[pl.ds(off, CHUNK)], idx_v)
            pltpu.async_copy(table_hbm.at[idx_v], rows_v, sem).wait()
            pltpu.sync_copy(rows_v, out_hbm.at[pl.ds(off, CHUNK)])

    return gather_kernel


@jax.jit
def kernel(lookup, table):
    B, H = lookup.shape
    V, D = table.shape
    N = B * H
    idx_flat = lookup.reshape(N).astype(jnp.int32)
    out = _build(N, V, D)(idx_flat, table)
    return out.reshape(B, H, D)


# speedup vs baseline: 1.1100x; 1.1100x over previous
import functools

import jax
import jax.numpy as jnp
from jax import lax
from jax.experimental import pallas as pl
from jax.experimental.pallas import tpu as pltpu
from jax.experimental.pallas import tpu_sc as plsc

NUM_CORES = 2
NUM_SUBCORES = 16
NW = NUM_CORES * NUM_SUBCORES

CHUNK = 3200


def _build(N, V, D):
    n_per_w = N // NW
    n_chunks = n_per_w // CHUNK
    assert n_per_w % CHUNK == 0

    mesh = plsc.VectorSubcoreMesh(core_axis_name="c", subcore_axis_name="s")

    @functools.partial(
        pl.kernel,
        out_type=jax.ShapeDtypeStruct((N, D), jnp.float32),
        mesh=mesh,
        scratch_types=[
            pltpu.VMEM((CHUNK,), jnp.int32),
            pltpu.VMEM((CHUNK, D), jnp.float32),
            pltpu.SemaphoreType.DMA,
        ],
        compiler_params=pltpu.CompilerParams(use_tc_tiling_on_sc=False),
    )
    def gather_kernel(idx_hbm, table_hbm, out_hbm, idx_v, rows_v, sem):
        wid = lax.axis_index("s") * NUM_CORES + lax.axis_index("c")
        base = wid * n_per_w

        @pl.loop(0, n_chunks)
        def _(i):
            off = base + i * CHUNK
            pltpu.sync_copy(idx_hbm.at[pl.ds(off, CHUNK)], idx_v)
            pltpu.async_copy(table_hbm.at[idx_v], rows_v, sem).wait()
            pltpu.sync_copy(rows_v, out_hbm.at[pl.ds(off, CHUNK)])

    return gather_kernel


@jax.jit
def kernel(lookup, table):
    B, H = lookup.shape
    V, D = table.shape
    N = B * H
    idx_flat = lookup.reshape(N).astype(jnp.int32)
    out = _build(N, V, D)(idx_flat, table)
    return out.reshape(B, H, D)


# speedup vs baseline: 1.1120x; 1.0018x over previous
import functools

import jax
import jax.numpy as jnp
from jax import lax
from jax.experimental import pallas as pl
from jax.experimental.pallas import tpu as pltpu
from jax.experimental.pallas import tpu_sc as plsc

NUM_CORES = 2
NUM_SUBCORES = 16
NW = NUM_CORES * NUM_SUBCORES

CHUNK = 640
NBUF = 4


def _build(N, V, D):
    n_per_w = N // NW
    n_chunks = n_per_w // CHUNK
    assert n_per_w % CHUNK == 0 and n_chunks % NBUF == 0 and n_chunks > NBUF

    mesh = plsc.VectorSubcoreMesh(core_axis_name="c", subcore_axis_name="s")

    @functools.partial(
        pl.kernel,
        out_type=jax.ShapeDtypeStruct((N, D), jnp.float32),
        mesh=mesh,
        scratch_types=[
            pltpu.VMEM((n_per_w,), jnp.int32),
            pltpu.VMEM((NBUF * CHUNK, D), jnp.float32),
            [pltpu.SemaphoreType.DMA] * NBUF,
            [pltpu.SemaphoreType.DMA] * NBUF,
        ],
        compiler_params=pltpu.CompilerParams(use_tc_tiling_on_sc=False),
    )
    def gather_kernel(idx_hbm, table_hbm, out_hbm, idx_v, rows_v, gsems, ssems):
        wid = lax.axis_index("s") * NUM_CORES + lax.axis_index("c")
        base = wid * n_per_w

        pltpu.sync_copy(idx_hbm.at[pl.ds(base, n_per_w)], idx_v)

        def gather_desc(chunk, b):
            return pltpu.make_async_copy(
                table_hbm.at[idx_v.at[pl.ds(chunk * CHUNK, CHUNK)]],
                rows_v.at[pl.ds(b * CHUNK, CHUNK)],
                gsems[b],
            )

        def store_desc(chunk, b):
            return pltpu.make_async_copy(
                rows_v.at[pl.ds(b * CHUNK, CHUNK)],
                out_hbm.at[pl.ds(base + chunk * CHUNK, CHUNK)],
                ssems[b],
            )

        for b in range(NBUF):
            gather_desc(b, b).start()

        @pl.loop(0, n_chunks - NBUF, step=NBUF)
        def _(i0):
            for b in range(NBUF):
                j = i0 + b
                gather_desc(j, b).wait()
                store_desc(j, b).start()
                store_desc(j, b).wait()
                gather_desc(j + NBUF, b).start()

        for b in range(NBUF):
            j = n_chunks - NBUF + b
            gather_desc(j, b).wait()
            store_desc(j, b).start()
        for b in range(NBUF):
            store_desc(n_chunks - NBUF + b, b).wait()

    return gather_kernel


@jax.jit
def kernel(lookup, table):
    B, H = lookup.shape
    V, D = table.shape
    N = B * H
    idx_flat = lookup.reshape(N).astype(jnp.int32)
    out = _build(N, V, D)(idx_flat, table)
    return out.reshape(B, H, D)


# speedup vs baseline: 1.6307x; 1.4664x over previous
import functools

import jax
import jax.numpy as jnp
from jax import lax
from jax.experimental import pallas as pl
from jax.experimental.pallas import tpu as pltpu
from jax.experimental.pallas import tpu_sc as plsc

NUM_CORES = 2
NUM_SUBCORES = 16
NW = NUM_CORES * NUM_SUBCORES


def _build(B, H, V, D):
    GD = D // 8
    CB = B // 128
    b_per_w = B // NW
    n_idx = b_per_w * H
    assert b_per_w == 512 and H == 50 and D == 32

    mesh = plsc.VectorSubcoreMesh(core_axis_name="c", subcore_axis_name="s")

    @functools.partial(
        pl.kernel,
        out_type=jax.ShapeDtypeStruct((H * GD * CB * 8 * 128,), jnp.float32),
        mesh=mesh,
        scratch_types=[
            pltpu.VMEM((n_idx,), jnp.int32),
            pltpu.VMEM((n_idx,), jnp.int32),
            pltpu.VMEM((2 * 512, 32), jnp.float32),
            pltpu.VMEM((2 * 16384,), jnp.float32),
            [pltpu.SemaphoreType.DMA] * 2,
            [pltpu.SemaphoreType.DMA] * 2,
        ],
        compiler_params=pltpu.CompilerParams(
            use_tc_tiling_on_sc=False, needs_layout_passes=False),
    )
    def gather_kernel(idx_hbm, table_hbm, x_hbm, slab, idxT, rows, xbuf,
                      gsems, ssems):
        wid = lax.axis_index("s") * NUM_CORES + lax.axis_index("c")
        iota = lax.iota(jnp.int32, 16)

        pltpu.sync_copy(idx_hbm.at[pl.ds(wid * n_idx, n_idx)], slab)

        v50 = iota * H

        @pl.loop(0, 32)
        def _(j0):
            for h in range(H):
                vals = plsc.load_gather(slab, [j0 * (16 * H) + v50 + h])
                idxT[pl.ds(h * 512 + j0 * 16, 16)] = vals

        def gather(h, p):
            return pltpu.make_async_copy(
                table_hbm.at[idxT.at[pl.ds(h * 512, 512)]],
                rows.at[pl.ds(p * 512, 512)],
                gsems[p],
            )

        def store(h, p, g):
            return pltpu.make_async_copy(
                xbuf.at[pl.ds(p * 16384 + g * 4096, 4096)],
                x_hbm.at[pl.ds((h * GD + g) * (CB * 1024) + wid * 4096, 4096)],
                ssems[p],
            )

        def transpose(h, p):
            @pl.loop(0, 32)
            def _(t):
                ridx = p * 512 + t * 16 + iota
                base = p * 16384 + (t // 8) * 1024 + (t % 8) * 16
                for g in range(GD):
                    for s in range(8):
                        col = jnp.full((16,), g * 8 + s, jnp.int32)
                        vals = plsc.load_gather(rows, [ridx, col])
                        xbuf[pl.ds(base + g * 4096 + s * 128, 16)] = vals

        def stage(h, p, first, last):
            gather(h, p).wait()
            if not first:
                for g in range(GD):
                    store(h - 2, p, g).wait()
            transpose(h, p)
            if not last:
                gather(h + 2, p).start()
            for g in range(GD):
                store(h, p, g).start()

        gather(0, 0).start()
        gather(1, 1).start()
        stage(0, 0, True, False)
        stage(1, 1, True, False)

        @pl.loop(2, H - 2, step=2)
        def _(h0):
            stage(h0, 0, False, False)
            stage(h0 + 1, 1, False, False)

        stage(H - 2, 0, False, True)
        stage(H - 1, 1, False, True)
        for g in range(GD):
            store(H - 2, 0, g).wait()
            store(H - 1, 1, g).wait()

    return gather_kernel


@jax.jit
def kernel(lookup, table):
    B, H = lookup.shape
    V, D = table.shape
    idx_flat = lookup.reshape(B * H).astype(jnp.int32)
    x = _build(B, H, V, D)(idx_flat, table)
    x5 = x.reshape(H, D // 8, B // 128, 8, 128)
    return jnp.transpose(x5, (2, 4, 0, 1, 3)).reshape(B, H, D)


# speedup vs baseline: 1.6323x; 1.0010x over previous
import functools

import jax
import jax.numpy as jnp
from jax import lax
from jax.experimental import pallas as pl
from jax.experimental.pallas import tpu as pltpu
from jax.experimental.pallas import tpu_sc as plsc

NUM_CORES = 2
NUM_SUBCORES = 16
NW = NUM_CORES * NUM_SUBCORES


def _build(B, H, V, D):
    GD = D // 8
    CB = B // 128
    b_per_w = B // NW
    n_idx = b_per_w * H
    assert b_per_w == 512 and H == 50 and D == 32

    mesh = plsc.VectorSubcoreMesh(core_axis_name="c", subcore_axis_name="s")

    @functools.partial(
        pl.kernel,
        out_type=jax.ShapeDtypeStruct((H * GD * CB * 8 * 128,), jnp.float32),
        mesh=mesh,
        scratch_types=[
            pltpu.VMEM((n_idx,), jnp.int32),
            pltpu.VMEM((n_idx,), jnp.int32),
            pltpu.VMEM((2 * 512, 32), jnp.float32),
            pltpu.VMEM((2 * 16384,), jnp.float32),
            [pltpu.SemaphoreType.DMA] * 2,
            [pltpu.SemaphoreType.DMA] * 2,
        ],
        compiler_params=pltpu.CompilerParams(
            use_tc_tiling_on_sc=False, needs_layout_passes=False,
            disable_bounds_checks=True),
    )
    def gather_kernel(idx_hbm, table_hbm, x_hbm, slab, idxT, rows, xbuf,
                      gsems, ssems):
        wid = lax.axis_index("s") * NUM_CORES + lax.axis_index("c")
        iota = lax.iota(jnp.int32, 16)

        pltpu.sync_copy(idx_hbm.at[pl.ds(wid * n_idx, n_idx)], slab)

        v50 = iota * H

        @pl.loop(0, 32)
        def _(j0):
            for h in range(H):
                vals = plsc.load_gather(slab, [j0 * (16 * H) + v50 + h])
                idxT[pl.ds(h * 512 + j0 * 16, 16)] = vals

        def gather(h, p):
            return pltpu.make_async_copy(
                table_hbm.at[idxT.at[pl.ds(h * 512, 512)]],
                rows.at[pl.ds(p * 512, 512)],
                gsems[p],
            )

        def store(h, p, g):
            return pltpu.make_async_copy(
                xbuf.at[pl.ds(p * 16384 + g * 4096, 4096)],
                x_hbm.at[pl.ds((h * GD + g) * (CB * 1024) + wid * 4096, 4096)],
                ssems[p],
            )

        cols = [jnp.full((16,), d, jnp.int32) for d in range(D)]

        def transpose(h, p):
            @pl.loop(0, 32)
            def _(t):
                ridx = p * 512 + t * 16 + iota
                base = p * 16384 + (t // 8) * 1024 + (t % 8) * 16
                for g in range(GD):
                    for s in range(8):
                        vals = plsc.load_gather(rows, [ridx, cols[g * 8 + s]])
                        xbuf[pl.ds(base + g * 4096 + s * 128, 16)] = vals

        def stage(h, p, first, last):
            gather(h, p).wait()
            if not first:
                for g in range(GD):
                    store(h - 2, p, g).wait()
            transpose(h, p)
            if not last:
                gather(h + 2, p).start()
            for g in range(GD):
                store(h, p, g).start()

        gather(0, 0).start()
        gather(1, 1).start()
        stage(0, 0, True, False)
        stage(1, 1, True, False)

        @pl.loop(2, H - 2, step=2)
        def _(h0):
            stage(h0, 0, False, False)
            stage(h0 + 1, 1, False, False)

        stage(H - 2, 0, False, True)
        stage(H - 1, 1, False, True)
        for g in range(GD):
            store(H - 2, 0, g).wait()
            store(H - 1, 1, g).wait()

    return gather_kernel


@jax.jit
def kernel(lookup, table):
    B, H = lookup.shape
    V, D = table.shape
    idx_flat = lookup.reshape(B * H).astype(jnp.int32)
    x = _build(B, H, V, D)(idx_flat, table)
    x5 = x.reshape(H, D // 8, B // 128, 8, 128)
    return jnp.transpose(x5, (2, 4, 0, 1, 3)).reshape(B, H, D)


# speedup vs baseline: 2.0690x; 1.2675x over previous
import functools

import jax
import jax.numpy as jnp
from jax import lax
from jax.experimental import pallas as pl
from jax.experimental.pallas import tpu as pltpu
from jax.experimental.pallas import tpu_sc as plsc

NUM_CORES = 2
NUM_SUBCORES = 16
NW = NUM_CORES * NUM_SUBCORES


def _build(B, H, V, D):
    GD = D // 8
    CB = B // 128
    b_per_w = B // NW
    n_idx = b_per_w * H
    assert b_per_w == 512 and H == 50 and D == 32

    mesh = plsc.VectorSubcoreMesh(core_axis_name="c", subcore_axis_name="s")

    @functools.partial(
        pl.kernel,
        out_type=jax.ShapeDtypeStruct((H * GD * CB * 8 * 128,), jnp.float32),
        mesh=mesh,
        scratch_types=[
            pltpu.VMEM((n_idx,), jnp.int32),
            pltpu.VMEM((n_idx,), jnp.int32),
            pltpu.VMEM((2 * 512, 32), jnp.float32),
            pltpu.VMEM((2 * 16384,), jnp.float32),
            [pltpu.SemaphoreType.DMA] * 2,
            [pltpu.SemaphoreType.DMA] * 2,
        ],
        compiler_params=pltpu.CompilerParams(
            use_tc_tiling_on_sc=False, needs_layout_passes=False,
            disable_bounds_checks=True),
    )
    def gather_kernel(idx_hbm, table_hbm, x_hbm, slab, idxT, rows, xbuf,
                      gsems, ssems):
        wid = lax.axis_index("s") * NUM_CORES + lax.axis_index("c")
        iota = lax.iota(jnp.int32, 16)

        pltpu.sync_copy(idx_hbm.at[pl.ds(wid * n_idx, n_idx)], slab)

        v50 = iota * H

        @pl.loop(0, 32)
        def _(j0):
            for h in range(H):
                vals = plsc.load_gather(slab, [j0 * (16 * H) + v50 + h])
                idxT[pl.ds(h * 512 + j0 * 16, 16)] = vals

        def gather(h, p):
            return pltpu.make_async_copy(
                table_hbm.at[idxT.at[pl.ds(h * 512, 512)]],
                rows.at[pl.ds(p * 512, 512)],
                gsems[p],
            )

        def store(h, p, g):
            return pltpu.make_async_copy(
                xbuf.at[pl.ds(p * 16384 + g * 4096, 4096)],
                x_hbm.at[pl.ds((h * GD + g) * (CB * 1024) + wid * 4096, 4096)],
                ssems[p],
            )

        cols = [jnp.full((16,), d, jnp.int32) for d in range(D)]

        def transpose(h, p):
            @plsc.parallel_loop(0, 32, unroll=2)
            def _(t):
                ridx = p * 512 + t * 16 + iota
                base = p * 16384 + (t // 8) * 1024 + (t % 8) * 16
                for g in range(GD):
                    for s in range(8):
                        vals = plsc.load_gather(rows, [ridx, cols[g * 8 + s]])
                        xbuf[pl.ds(base + g * 4096 + s * 128, 16)] = vals

        def stage(h, p, first, last):
            gather(h, p).wait()
            if not first:
                for g in range(GD):
                    store(h - 2, p, g).wait()
            transpose(h, p)
            if not last:
                gather(h + 2, p).start()
            for g in range(GD):
                store(h, p, g).start()

        gather(0, 0).start()
        gather(1, 1).start()
        stage(0, 0, True, False)
        stage(1, 1, True, False)

        @pl.loop(2, H - 2, step=2)
        def _(h0):
            stage(h0, 0, False, False)
            stage(h0 + 1, 1, False, False)

        stage(H - 2, 0, False, True)
        stage(H - 1, 1, False, True)
        for g in range(GD):
            store(H - 2, 0, g).wait()
            store(H - 1, 1, g).wait()

    return gather_kernel


@jax.jit
def kernel(lookup, table):
    B, H = lookup.shape
    V, D = table.shape
    idx_flat = lookup.reshape(B * H).astype(jnp.int32)
    x = _build(B, H, V, D)(idx_flat, table)
    x5 = x.reshape(H, D // 8, B // 128, 8, 128)
    return jnp.transpose(x5, (2, 4, 0, 1, 3)).reshape(B, H, D)
